# baseline (speedup 1.0000x reference)
import functools

import jax
import jax.numpy as jnp
from jax import lax
from jax.experimental import pallas as pl
from jax.experimental.pallas import tpu as pltpu
from jax.experimental.pallas import tpu_sc as plsc

NC = 2
NS = 16
NW = NC * NS

IB = 128
ROWS_PER_CHUNK = 5
CHUNK = IB * ROWS_PER_CHUNK


def _sc_body(ids_ref, comb_ref, gate_ref, out_ref,
             idx_v, buf_v, out_v, gate_v, sem, *, ids_per_worker, d):
    wid = lax.axis_index("s") * NC + lax.axis_index("c")
    id0 = wid * ids_per_worker
    n_chunks = ids_per_worker // CHUNK

    pltpu.sync_copy(ids_ref.at[pl.ds(id0, ids_per_worker)], idx_v)

    dnums = lax.GatherDimensionNumbers(
        offset_dims=(), collapsed_slice_dims=(0,), start_index_map=(0,))

    for c in range(n_chunks):
        copies = []
        for j in range(ROWS_PER_CHUNK):
            idx_row = idx_v.at[pl.ds(c * CHUNK + j * IB, IB)]
            dst = pl.ds(j * IB, IB)
            copies.append(pltpu.async_copy(comb_ref.at[idx_row], buf_v.at[dst], sem))
            copies.append(pltpu.async_copy(gate_ref.at[idx_row], gate_v.at[dst], sem))
        for cp in copies:
            cp.wait()

        def combine(grp, _):
            g16 = gate_v[pl.ds(grp * 16, 16)]
            w16 = 1.0 / (1.0 + jnp.exp(-g16))
            for j in range(16):
                row = grp * 16 + j
                w = lax.gather(
                    w16, jnp.full((16, 1), j, jnp.int32), dnums,
                    slice_sizes=(1,),
                    mode=lax.GatherScatterMode.PROMISE_IN_BOUNDS)
                for k in range(d // 16):
                    out_v[row, pl.ds(k * 16, 16)] = (
                        buf_v[row, pl.ds(k * 16, 16)]
                        + w * buf_v[row, pl.ds(d + k * 16, 16)])
            return 0

        lax.fori_loop(0, CHUNK // 16, combine, 0)

        pltpu.sync_copy(out_v, out_ref.at[pl.ds(id0 + c * CHUNK, CHUNK)])


def kernel(input_ids, base_weight, prior_matrix, gate_logits):
    b, l = input_ids.shape
    v, d = base_weight.shape
    n = b * l
    assert n % (NW * CHUNK) == 0 and d % 16 == 0
    ids_per_worker = n // NW

    ids1 = input_ids.reshape(n)
    comb = jnp.concatenate([base_weight, prior_matrix], axis=1)

    mesh = plsc.VectorSubcoreMesh(core_axis_name="c", subcore_axis_name="s")
    body = functools.partial(_sc_body, ids_per_worker=ids_per_worker, d=d)
    call = pl.kernel(
        body,
        mesh=mesh,
        compiler_params=pltpu.CompilerParams(use_tc_tiling_on_sc=False),
        out_type=jax.ShapeDtypeStruct((n, d), jnp.float32),
        scratch_types=[
            pltpu.VMEM((ids_per_worker,), jnp.int32),
            pltpu.VMEM((CHUNK, 2 * d), jnp.float32),
            pltpu.VMEM((CHUNK, d), jnp.float32),
            pltpu.VMEM((CHUNK,), jnp.float32),
            pltpu.SemaphoreType.DMA,
        ],
    )
    out = call(ids1, comb, gate_logits)
    return out.reshape(b, l, d)


# speedup vs baseline: 1.0006x; 1.0006x over previous
import functools

import jax
import jax.numpy as jnp
from jax import lax
from jax.experimental import pallas as pl
from jax.experimental.pallas import tpu as pltpu
from jax.experimental.pallas import tpu_sc as plsc

NC = 2
NS = 16
NW = NC * NS

IB = 128
ROWS_PER_CHUNK = 5
CHUNK = IB * ROWS_PER_CHUNK


def _sc_body(ids_ref, comb_ref, gate_ref, out_ref,
             idx_v, buf_v, out_v, gate_v, sem, *, ids_per_worker, d):
    wid = lax.axis_index("s") * NC + lax.axis_index("c")
    id0 = wid * ids_per_worker
    n_chunks = ids_per_worker // CHUNK

    pltpu.sync_copy(ids_ref.at[pl.ds(id0, ids_per_worker)], idx_v)

    dnums = lax.GatherDimensionNumbers(
        offset_dims=(), collapsed_slice_dims=(0,), start_index_map=(0,))

    for c in range(n_chunks):
        copies = []
        for j in range(ROWS_PER_CHUNK):
            idx_row = idx_v.at[pl.ds(c * CHUNK + j * IB, IB)]
            dst = pl.ds(j * IB, IB)
            for h in range(2):
                idx_h = idx_v.at[pl.ds(c * CHUNK + j * IB + h * (IB // 2), IB // 2)]
                dst_h = pl.ds(j * IB + h * (IB // 2), IB // 2)
                copies.append(pltpu.async_copy(comb_ref.at[idx_h], buf_v.at[dst_h], sem))
            copies.append(pltpu.async_copy(gate_ref.at[idx_row], gate_v.at[dst], sem))
        for cp in copies:
            cp.wait()

        def combine(grp, _):
            g16 = gate_v[pl.ds(grp * 16, 16)]
            w16 = 1.0 / (1.0 + jnp.exp(-g16))
            for j in range(16):
                row = grp * 16 + j
                w = lax.gather(
                    w16, jnp.full((16, 1), j, jnp.int32), dnums,
                    slice_sizes=(1,),
                    mode=lax.GatherScatterMode.PROMISE_IN_BOUNDS)
                for k in range(d // 16):
                    out_v[row, pl.ds(k * 16, 16)] = (
                        buf_v[row, pl.ds(k * 16, 16)]
                        + w * buf_v[row, pl.ds(d + k * 16, 16)])
            return 0

        lax.fori_loop(0, CHUNK // 16, combine, 0)

        pltpu.sync_copy(out_v, out_ref.at[pl.ds(id0 + c * CHUNK, CHUNK)])


def kernel(input_ids, base_weight, prior_matrix, gate_logits):
    b, l = input_ids.shape
    v, d = base_weight.shape
    n = b * l
    assert n % (NW * CHUNK) == 0 and d % 16 == 0
    ids_per_worker = n // NW

    ids1 = input_ids.reshape(n)
    comb = jnp.concatenate([base_weight, prior_matrix], axis=1)

    mesh = plsc.VectorSubcoreMesh(core_axis_name="c", subcore_axis_name="s")
    body = functools.partial(_sc_body, ids_per_worker=ids_per_worker, d=d)
    call = pl.kernel(
        body,
        mesh=mesh,
        compiler_params=pltpu.CompilerParams(use_tc_tiling_on_sc=False),
        out_type=jax.ShapeDtypeStruct((n, d), jnp.float32),
        scratch_types=[
            pltpu.VMEM((ids_per_worker,), jnp.int32),
            pltpu.VMEM((CHUNK, 2 * d), jnp.float32),
            pltpu.VMEM((CHUNK, d), jnp.float32),
            pltpu.VMEM((CHUNK,), jnp.float32),
            pltpu.SemaphoreType.DMA,
        ],
    )
    out = call(ids1, comb, gate_logits)
    return out.reshape(b, l, d)


# speedup vs baseline: 1.3479x; 1.3471x over previous
import functools

import jax
import jax.numpy as jnp
from jax import lax
from jax.experimental import pallas as pl
from jax.experimental.pallas import tpu as pltpu
from jax.experimental.pallas import tpu_sc as plsc

NC = 2
NS = 16
NW = NC * NS

CHUNK = 128


def _sc_body(ids_ref, base_ref, prior_ref, gate_ref, out_ref,
             idx_v, base_a, base_b, prior_a, prior_b, gate_a, gate_b,
             sem_a, sem_b, *, ids_per_worker, d):
    wid = lax.axis_index("s") * NC + lax.axis_index("c")
    id0 = wid * ids_per_worker
    n_chunks = ids_per_worker // CHUNK
    n_pairs = n_chunks // 2

    pltpu.sync_copy(ids_ref.at[pl.ds(id0, ids_per_worker)], idx_v)

    dnums = lax.GatherDimensionNumbers(
        offset_dims=(), collapsed_slice_dims=(0,), start_index_map=(0,))

    def fire(c, base_v, prior_v, gate_v, sem):
        idx = idx_v.at[pl.ds(c * CHUNK, CHUNK)]
        pltpu.async_copy(base_ref.at[idx], base_v, sem)
        pltpu.async_copy(prior_ref.at[idx], prior_v, sem)
        pltpu.async_copy(gate_ref.at[idx], gate_v, sem)

    def wait(base_v, prior_v, gate_v, sem):
        pltpu.make_async_copy(base_ref.at[pl.ds(0, CHUNK)], base_v, sem).wait()
        pltpu.make_async_copy(prior_ref.at[pl.ds(0, CHUNK)], prior_v, sem).wait()
        pltpu.make_async_copy(gate_ref.at[pl.ds(0, CHUNK)], gate_v, sem).wait()

    def combine(base_v, prior_v, gate_v):
        def grp_body(grp, _):
            g16 = gate_v[pl.ds(grp * 16, 16)]
            w16 = 1.0 / (1.0 + jnp.exp(-g16))
            for j in range(16):
                row = grp * 16 + j
                w = lax.gather(
                    w16, jnp.full((16, 1), j, jnp.int32), dnums,
                    slice_sizes=(1,),
                    mode=lax.GatherScatterMode.PROMISE_IN_BOUNDS)
                for k in range(d // 16):
                    sl = pl.ds(k * 16, 16)
                    prior_v[row, sl] = base_v[row, sl] + w * prior_v[row, sl]
            return 0

        lax.fori_loop(0, CHUNK // 16, grp_body, 0)

    def writeback(c, prior_v):
        off = pl.multiple_of(id0 + c * CHUNK, 8)
        pltpu.sync_copy(prior_v, out_ref.at[pl.ds(off, CHUNK)])

    fire(0, base_a, prior_a, gate_a, sem_a)

    def pair_body(t, _):
        ca = 2 * t
        wait(base_a, prior_a, gate_a, sem_a)
        fire(ca + 1, base_b, prior_b, gate_b, sem_b)
        combine(base_a, prior_a, gate_a)
        writeback(ca, prior_a)
        wait(base_b, prior_b, gate_b, sem_b)

        @pl.when(t < n_pairs - 1)
        def _():
            fire(ca + 2, base_a, prior_a, gate_a, sem_a)

        combine(base_b, prior_b, gate_b)
        writeback(ca + 1, prior_b)
        return 0

    lax.fori_loop(0, n_pairs, pair_body, 0)


def kernel(input_ids, base_weight, prior_matrix, gate_logits):
    b, l = input_ids.shape
    v, d = base_weight.shape
    n = b * l
    assert n % (NW * 2 * CHUNK) == 0 and d % 16 == 0
    ids_per_worker = n // NW

    ids1 = input_ids.reshape(n)

    mesh = plsc.VectorSubcoreMesh(core_axis_name="c", subcore_axis_name="s")
    body = functools.partial(_sc_body, ids_per_worker=ids_per_worker, d=d)
    call = pl.kernel(
        body,
        mesh=mesh,
        compiler_params=pltpu.CompilerParams(use_tc_tiling_on_sc=False),
        out_type=jax.ShapeDtypeStruct((n, d), jnp.float32),
        scratch_types=[
            pltpu.VMEM((ids_per_worker,), jnp.int32),
            pltpu.VMEM((CHUNK, d), jnp.float32),
            pltpu.VMEM((CHUNK, d), jnp.float32),
            pltpu.VMEM((CHUNK, d), jnp.float32),
            pltpu.VMEM((CHUNK, d), jnp.float32),
            pltpu.VMEM((CHUNK,), jnp.float32),
            pltpu.VMEM((CHUNK,), jnp.float32),
            pltpu.SemaphoreType.DMA,
            pltpu.SemaphoreType.DMA,
        ],
    )
    out = call(ids1, base_weight, prior_matrix, gate_logits)
    return out.reshape(b, l, d)


# speedup vs baseline: 1.8066x; 1.3403x over previous
import functools

import jax
import jax.numpy as jnp
from jax import lax
from jax.experimental import pallas as pl
from jax.experimental.pallas import tpu as pltpu
from jax.experimental.pallas import tpu_sc as plsc

NC = 2
NS = 16
NW = NC * NS

RPC = 8
GROUPS = ((0, 0), (0, 16), (0, 32), (0, 34),)


def _sc_body(ids_ref, base_ref, prior_ref, gate_ref, out_ref,
             idx_a, idx_b, base_a, base_b, prior_a, prior_b, gate_a, gate_b,
             out_v, sem_a, sem_b, *, rows_per_worker, l, d):
    wid = lax.axis_index("s") * NC + lax.axis_index("c")
    chunk = RPC * l
    row0 = wid * rows_per_worker
    id0 = row0 * l
    n_chunks = rows_per_worker // RPC
    n_pairs = n_chunks // 2

    dnums = lax.GatherDimensionNumbers(
        offset_dims=(), collapsed_slice_dims=(0,), start_index_map=(0,))

    SEGS = [(0, 128), (128, 128), (256, 128), (384, 16)]

    def fire(c, idx_v, base_v, prior_v, gate_v, sem):
        pltpu.sync_copy(ids_ref.at[pl.ds(id0 + c * chunk, chunk)], idx_v)
        for off, ln in SEGS:
            idx = idx_v.at[pl.ds(off, ln)]
            pltpu.async_copy(base_ref.at[idx], base_v.at[pl.ds(off, ln)], sem)
            pltpu.async_copy(prior_ref.at[idx], prior_v.at[pl.ds(off, ln)], sem)
            pltpu.async_copy(gate_ref.at[idx], gate_v.at[pl.ds(off, ln)], sem)

    def wait(base_v, prior_v, gate_v, sem):
        for off, ln in SEGS:
            pltpu.make_async_copy(
                base_ref.at[pl.ds(0, ln)], base_v.at[pl.ds(off, ln)], sem).wait()
            pltpu.make_async_copy(
                prior_ref.at[pl.ds(0, ln)], prior_v.at[pl.ds(off, ln)], sem).wait()
            pltpu.make_async_copy(
                gate_ref.at[pl.ds(0, ln)], gate_v.at[pl.ds(off, ln)], sem).wait()

    def combine(base_v, prior_v, gate_v):
        def q_body(q, _):
            r0 = q * l
            for _, lo in GROUPS:
                g16 = gate_v[pl.ds(r0 + lo, 16)]
                w16 = 1.0 / (1.0 + jnp.exp(-g16))
                for j in range(16):
                    row = r0 + lo + j
                    w = lax.gather(
                        w16, jnp.full((16, 1), j, jnp.int32), dnums,
                        slice_sizes=(1,),
                        mode=lax.GatherScatterMode.PROMISE_IN_BOUNDS)
                    for k in range(d // 16):
                        sl = pl.ds(k * 16, 16)
                        out_v[q, lo + j, sl] = (
                            base_v[row, sl] + w * prior_v[row, sl])
            return 0

        lax.fori_loop(0, RPC, q_body, 0)

    def writeback(c):
        off = pl.multiple_of(row0 + c * RPC, 8)
        pltpu.sync_copy(out_v, out_ref.at[pl.ds(off, RPC), pl.ds(0, l), pl.ds(0, d)])

    fire(0, idx_a, base_a, prior_a, gate_a, sem_a)

    def pair_body(t, _):
        ca = 2 * t
        wait(base_a, prior_a, gate_a, sem_a)
        fire(ca + 1, idx_b, base_b, prior_b, gate_b, sem_b)
        combine(base_a, prior_a, gate_a)
        writeback(ca)
        wait(base_b, prior_b, gate_b, sem_b)

        @pl.when(t < n_pairs - 1)
        def _():
            fire(ca + 2, idx_a, base_a, prior_a, gate_a, sem_a)

        combine(base_b, prior_b, gate_b)
        writeback(ca + 1)
        return 0

    lax.fori_loop(0, n_pairs, pair_body, 0)


def kernel(input_ids, base_weight, prior_matrix, gate_logits):
    b, l = input_ids.shape
    v, d = base_weight.shape
    n = b * l
    assert b % (NW * 2 * RPC) == 0 and d % 16 == 0 and l == 50
    rows_per_worker = b // NW

    ids1 = input_ids.reshape(n)

    mesh = plsc.VectorSubcoreMesh(core_axis_name="c", subcore_axis_name="s")
    body = functools.partial(_sc_body, rows_per_worker=rows_per_worker, l=l, d=d)
    chunk = RPC * l
    call = pl.kernel(
        body,
        mesh=mesh,
        compiler_params=pltpu.CompilerParams(use_tc_tiling_on_sc=False),
        out_type=jax.ShapeDtypeStruct((b, 56, 128), jnp.float32),
        scratch_types=[
            pltpu.VMEM((chunk,), jnp.int32),
            pltpu.VMEM((chunk,), jnp.int32),
            pltpu.VMEM((chunk, d), jnp.float32),
            pltpu.VMEM((chunk, d), jnp.float32),
            pltpu.VMEM((chunk, d), jnp.float32),
            pltpu.VMEM((chunk, d), jnp.float32),
            pltpu.VMEM((chunk,), jnp.float32),
            pltpu.VMEM((chunk,), jnp.float32),
            pltpu.VMEM((RPC, l, d), jnp.float32),
            pltpu.SemaphoreType.DMA,
            pltpu.SemaphoreType.DMA,
        ],
    )
    out = call(ids1, base_weight, prior_matrix, gate_logits)
    return out[:, :l, :d]


# speedup vs baseline: 1.8454x; 1.0215x over previous
import functools

import jax
import jax.numpy as jnp
from jax import lax
from jax.experimental import pallas as pl
from jax.experimental.pallas import tpu as pltpu
from jax.experimental.pallas import tpu_sc as plsc

NC = 2
NS = 16
NW = NC * NS

RPC = 8
GROUPS = ((0, 0), (0, 16), (0, 32), (0, 34),)


def _sc_body(ids_ref, base_ref, prior_ref, gate_ref, out_ref,
             idx_a, idx_b, base_a, base_b, prior_a, prior_b, gate_a, gate_b,
             out_v, sem_a, sem_b, *, rows_per_worker, l, d):
    wid = lax.axis_index("s") * NC + lax.axis_index("c")
    chunk = RPC * l
    row0 = wid * rows_per_worker
    id0 = row0 * l
    n_chunks = rows_per_worker // RPC
    n_pairs = n_chunks // 2

    dnums = lax.GatherDimensionNumbers(
        offset_dims=(), collapsed_slice_dims=(0,), start_index_map=(0,))

    SEGS = [(0, 128), (128, 128), (256, 128), (384, 16)]

    def fire(c, idx_v, base_v, prior_v, gate_v, sem):
        pltpu.sync_copy(ids_ref.at[pl.ds(id0 + c * chunk, chunk)], idx_v)
        for off, ln in SEGS:
            idx = idx_v.at[pl.ds(off, ln)]
            pltpu.async_copy(base_ref.at[idx], base_v.at[pl.ds(off, ln)], sem)
            pltpu.async_copy(prior_ref.at[idx], prior_v.at[pl.ds(off, ln)], sem)
            pltpu.async_copy(gate_ref.at[idx], gate_v.at[pl.ds(off, ln)], sem)

    def wait(base_v, prior_v, gate_v, sem):
        for off, ln in SEGS:
            pltpu.make_async_copy(
                base_ref.at[pl.ds(0, ln)], base_v.at[pl.ds(off, ln)], sem).wait()
            pltpu.make_async_copy(
                prior_ref.at[pl.ds(0, ln)], prior_v.at[pl.ds(off, ln)], sem).wait()
            pltpu.make_async_copy(
                gate_ref.at[pl.ds(0, ln)], gate_v.at[pl.ds(off, ln)], sem).wait()

    def combine(base_v, prior_v, gate_v):
        def q_body(q, _):
            r0 = q * l
            for lo, js in ((0, range(16)), (16, range(16)), (32, range(16)),
                           (34, (14, 15))):
                g16 = gate_v[pl.ds(r0 + lo, 16)]
                w16 = 1.0 / (1.0 + jnp.exp(-g16))
                for j in js:
                    row = r0 + lo + j
                    w = lax.gather(
                        w16, jnp.full((16, 1), j, jnp.int32), dnums,
                        slice_sizes=(1,),
                        mode=lax.GatherScatterMode.PROMISE_IN_BOUNDS)
                    for k in range(d // 16):
                        sl = pl.ds(k * 16, 16)
                        out_v[q, lo + j, sl] = (
                            base_v[row, sl] + w * prior_v[row, sl])
            return 0

        lax.fori_loop(0, RPC, q_body, 0)

    def writeback(c):
        off = pl.multiple_of(row0 + c * RPC, 8)
        pltpu.sync_copy(out_v, out_ref.at[pl.ds(off, RPC), pl.ds(0, l), pl.ds(0, d)])

    fire(0, idx_a, base_a, prior_a, gate_a, sem_a)

    def pair_body(t, _):
        ca = 2 * t
        wait(base_a, prior_a, gate_a, sem_a)
        fire(ca + 1, idx_b, base_b, prior_b, gate_b, sem_b)
        combine(base_a, prior_a, gate_a)
        writeback(ca)
        wait(base_b, prior_b, gate_b, sem_b)

        @pl.when(t < n_pairs - 1)
        def _():
            fire(ca + 2, idx_a, base_a, prior_a, gate_a, sem_a)

        combine(base_b, prior_b, gate_b)
        writeback(ca + 1)
        return 0

    lax.fori_loop(0, n_pairs, pair_body, 0)


def kernel(input_ids, base_weight, prior_matrix, gate_logits):
    b, l = input_ids.shape
    v, d = base_weight.shape
    n = b * l
    assert b % (NW * 2 * RPC) == 0 and d % 16 == 0 and l == 50
    rows_per_worker = b // NW

    ids1 = input_ids.reshape(n)

    mesh = plsc.VectorSubcoreMesh(core_axis_name="c", subcore_axis_name="s")
    body = functools.partial(_sc_body, rows_per_worker=rows_per_worker, l=l, d=d)
    chunk = RPC * l
    call = pl.kernel(
        body,
        mesh=mesh,
        compiler_params=pltpu.CompilerParams(use_tc_tiling_on_sc=False),
        out_type=jax.ShapeDtypeStruct((b, 56, 128), jnp.float32),
        scratch_types=[
            pltpu.VMEM((chunk,), jnp.int32),
            pltpu.VMEM((chunk,), jnp.int32),
            pltpu.VMEM((chunk, d), jnp.float32),
            pltpu.VMEM((chunk, d), jnp.float32),
            pltpu.VMEM((chunk, d), jnp.float32),
            pltpu.VMEM((chunk, d), jnp.float32),
            pltpu.VMEM((chunk,), jnp.float32),
            pltpu.VMEM((chunk,), jnp.float32),
            pltpu.VMEM((RPC, l, d), jnp.float32),
            pltpu.SemaphoreType.DMA,
            pltpu.SemaphoreType.DMA,
        ],
    )
    out = call(ids1, base_weight, prior_matrix, gate_logits)
    return out[:, :l, :d]


# speedup vs baseline: 1.8542x; 1.0048x over previous
import functools

import jax
import jax.numpy as jnp
from jax import lax
from jax.experimental import pallas as pl
from jax.experimental.pallas import tpu as pltpu
from jax.experimental.pallas import tpu_sc as plsc

NC = 2
NS = 16
NW = NC * NS

RPC = 4
GROUPS = ((0, 0), (0, 16), (0, 32), (0, 34),)


def _sc_body(ids_ref, base_ref, prior_ref, gate_ref, out_ref,
             idx_v, base_a, base_b, prior_a, prior_b, gate_a, gate_b,
             out_v, sem_a, sem_b, *, rows_per_worker, l, d):
    wid = lax.axis_index("s") * NC + lax.axis_index("c")
    chunk = RPC * l
    row0 = wid * rows_per_worker
    id0 = row0 * l
    n_chunks = rows_per_worker // RPC
    n_pairs = n_chunks // 2

    pltpu.sync_copy(ids_ref.at[pl.ds(id0, rows_per_worker * l)], idx_v)

    dnums = lax.GatherDimensionNumbers(
        offset_dims=(), collapsed_slice_dims=(0,), start_index_map=(0,))

    SEGS = [(0, 128), (128, 72)]

    def fire(c, base_v, prior_v, gate_v, sem):
        for off, ln in SEGS:
            idx = idx_v.at[pl.ds(c * chunk + off, ln)]
            pltpu.async_copy(base_ref.at[idx], base_v.at[pl.ds(off, ln)], sem)
            pltpu.async_copy(prior_ref.at[idx], prior_v.at[pl.ds(off, ln)], sem)
            pltpu.async_copy(gate_ref.at[idx], gate_v.at[pl.ds(off, ln)], sem)

    def wait(base_v, prior_v, gate_v, sem):
        for off, ln in SEGS:
            pltpu.make_async_copy(
                base_ref.at[pl.ds(0, ln)], base_v.at[pl.ds(off, ln)], sem).wait()
            pltpu.make_async_copy(
                prior_ref.at[pl.ds(0, ln)], prior_v.at[pl.ds(off, ln)], sem).wait()
            pltpu.make_async_copy(
                gate_ref.at[pl.ds(0, ln)], gate_v.at[pl.ds(off, ln)], sem).wait()

    def combine(base_v, prior_v, gate_v):
        def q_body(q, _):
            r0 = q * l
            for lo, js in ((0, range(16)), (16, range(16)), (32, range(16)),
                           (34, (14, 15))):
                g16 = gate_v[pl.ds(r0 + lo, 16)]
                w16 = 1.0 / (1.0 + jnp.exp(-g16))
                for j in js:
                    row = r0 + lo + j
                    w = lax.gather(
                        w16, jnp.full((16, 1), j, jnp.int32), dnums,
                        slice_sizes=(1,),
                        mode=lax.GatherScatterMode.PROMISE_IN_BOUNDS)
                    for k in range(d // 16):
                        sl = pl.ds(k * 16, 16)
                        out_v[q, lo + j, sl] = (
                            base_v[row, sl] + w * prior_v[row, sl])
            return 0

        lax.fori_loop(0, RPC, q_body, 0)

    def writeback(c):
        off = row0 + c * RPC
        pltpu.sync_copy(out_v, out_ref.at[pl.ds(off, RPC), pl.ds(0, l), pl.ds(0, d)])

    fire(0, base_a, prior_a, gate_a, sem_a)

    def pair_body(t, _):
        ca = 2 * t
        wait(base_a, prior_a, gate_a, sem_a)
        fire(ca + 1, base_b, prior_b, gate_b, sem_b)
        combine(base_a, prior_a, gate_a)
        writeback(ca)
        wait(base_b, prior_b, gate_b, sem_b)

        @pl.when(t < n_pairs - 1)
        def _():
            fire(ca + 2, base_a, prior_a, gate_a, sem_a)

        combine(base_b, prior_b, gate_b)
        writeback(ca + 1)
        return 0

    lax.fori_loop(0, n_pairs, pair_body, 0)


def kernel(input_ids, base_weight, prior_matrix, gate_logits):
    b, l = input_ids.shape
    v, d = base_weight.shape
    n = b * l
    assert b % (NW * 2 * RPC) == 0 and d % 16 == 0 and l == 50
    rows_per_worker = b // NW

    ids1 = input_ids.reshape(n)

    mesh = plsc.VectorSubcoreMesh(core_axis_name="c", subcore_axis_name="s")
    body = functools.partial(_sc_body, rows_per_worker=rows_per_worker, l=l, d=d)
    chunk = RPC * l
    call = pl.kernel(
        body,
        mesh=mesh,
        compiler_params=pltpu.CompilerParams(use_tc_tiling_on_sc=False),
        out_type=jax.ShapeDtypeStruct((b, 56, 128), jnp.float32),
        scratch_types=[
            pltpu.VMEM((rows_per_worker * l,), jnp.int32),
            pltpu.VMEM((chunk, d), jnp.float32),
            pltpu.VMEM((chunk, d), jnp.float32),
            pltpu.VMEM((chunk, d), jnp.float32),
            pltpu.VMEM((chunk, d), jnp.float32),
            pltpu.VMEM((chunk,), jnp.float32),
            pltpu.VMEM((chunk,), jnp.float32),
            pltpu.VMEM((RPC, l, d), jnp.float32),
            pltpu.SemaphoreType.DMA,
            pltpu.SemaphoreType.DMA,
        ],
    )
    out = call(ids1, base_weight, prior_matrix, gate_logits)
    return out[:, :l, :d]
